# baseline (speedup 1.0000x reference)
import jax
import jax.numpy as jnp
from jax.experimental import pallas as pl
from jax.experimental.pallas import tpu as pltpu

_ROWS, _D = 32768, 2048
_CHUNK = 512
_N = _ROWS // _CHUNK
_SLOTS = 8
_RAHEAD = 5
_WBEHIND = 3


def _copy_body(x_hbm, o_hbm, *scratch):
    bufs = scratch[:_SLOTS]
    rsems = scratch[_SLOTS:2 * _SLOTS]
    wsems = scratch[2 * _SLOTS:]

    def rd(j):
        s = j % _SLOTS
        return pltpu.make_async_copy(
            x_hbm.at[pl.ds(j * _CHUNK, _CHUNK)], bufs[s], rsems[s])

    def wr(j):
        s = j % _SLOTS
        return pltpu.make_async_copy(
            bufs[s], o_hbm.at[pl.ds(j * _CHUNK, _CHUNK)], wsems[s])

    rh = [None] * _N
    wh = [None] * _N
    for j in range(min(_RAHEAD, _N)):
        rh[j] = rd(j)
        rh[j].start()
    for j in range(_N):
        rh[j].wait()
        wh[j] = wr(j)
        wh[j].start()
        if j - _WBEHIND >= 0:
            wh[j - _WBEHIND].wait()
        if j + _RAHEAD < _N:
            rh[j + _RAHEAD] = rd(j + _RAHEAD)
            rh[j + _RAHEAD].start()
    for j in range(max(0, _N - _WBEHIND), _N):
        wh[j].wait()


def kernel(hidden_state):
    B, S, D = hidden_state.shape
    x = hidden_state.reshape(B * S, D)
    out = pl.pallas_call(
        _copy_body,
        in_specs=[pl.BlockSpec(memory_space=pl.ANY)],
        out_specs=pl.BlockSpec(memory_space=pl.ANY),
        out_shape=jax.ShapeDtypeStruct((B * S, D), hidden_state.dtype),
        scratch_shapes=(
            [pltpu.VMEM((_CHUNK, _D), jnp.float32)] * _SLOTS
            + [pltpu.SemaphoreType.DMA] * (2 * _SLOTS)
        ),
        compiler_params=pltpu.CompilerParams(
            vmem_limit_bytes=100 * 1024 * 1024,
        ),
    )(x)
    return out.reshape(B, S, D)


# speedup vs baseline: 1.0034x; 1.0034x over previous
import jax
import jax.numpy as jnp
from jax.experimental import pallas as pl
from jax.experimental.pallas import tpu as pltpu

_ROWS, _D = 32768, 2048
_CHUNK = 2048
_N = _ROWS // _CHUNK
_SLOTS = 3
_RAHEAD = 2
_WBEHIND = 1


def _copy_body(x_hbm, o_hbm, *scratch):
    bufs = scratch[:_SLOTS]
    rsems = scratch[_SLOTS:2 * _SLOTS]
    wsems = scratch[2 * _SLOTS:]

    def rd(j):
        s = j % _SLOTS
        return pltpu.make_async_copy(
            x_hbm.at[pl.ds(j * _CHUNK, _CHUNK)], bufs[s], rsems[s])

    def wr(j):
        s = j % _SLOTS
        return pltpu.make_async_copy(
            bufs[s], o_hbm.at[pl.ds(j * _CHUNK, _CHUNK)], wsems[s])

    rh = [None] * _N
    wh = [None] * _N
    for j in range(min(_RAHEAD, _N)):
        rh[j] = rd(j)
        rh[j].start()
    for j in range(_N):
        rh[j].wait()
        wh[j] = wr(j)
        wh[j].start()
        if j - _WBEHIND >= 0:
            wh[j - _WBEHIND].wait()
        if j + _RAHEAD < _N:
            rh[j + _RAHEAD] = rd(j + _RAHEAD)
            rh[j + _RAHEAD].start()
    for j in range(max(0, _N - _WBEHIND), _N):
        wh[j].wait()


def kernel(hidden_state):
    B, S, D = hidden_state.shape
    x = hidden_state.reshape(B * S, D)
    out = pl.pallas_call(
        _copy_body,
        in_specs=[pl.BlockSpec(memory_space=pl.ANY)],
        out_specs=pl.BlockSpec(memory_space=pl.ANY),
        out_shape=jax.ShapeDtypeStruct((B * S, D), hidden_state.dtype),
        scratch_shapes=(
            [pltpu.VMEM((_CHUNK, _D), jnp.float32)] * _SLOTS
            + [pltpu.SemaphoreType.DMA] * (2 * _SLOTS)
        ),
        compiler_params=pltpu.CompilerParams(
            vmem_limit_bytes=100 * 1024 * 1024,
        ),
    )(x)
    return out.reshape(B, S, D)


# speedup vs baseline: 1.0037x; 1.0003x over previous
import jax
import jax.numpy as jnp
from jax.experimental import pallas as pl
from jax.experimental.pallas import tpu as pltpu


def _copy_body(x_ref, o_ref):
    o_ref[...] = x_ref[...]


def kernel(hidden_state):
    B, S, D = hidden_state.shape
    x = hidden_state.reshape(B * S, D)
    rows = B * S
    block_rows = 2040
    grid = (pl.cdiv(rows, block_rows),)
    out = pl.pallas_call(
        _copy_body,
        grid=grid,
        in_specs=[pl.BlockSpec((block_rows, D), lambda i: (i, 0))],
        out_specs=pl.BlockSpec((block_rows, D), lambda i: (i, 0)),
        out_shape=jax.ShapeDtypeStruct((rows, D), hidden_state.dtype),
        compiler_params=pltpu.CompilerParams(
            vmem_limit_bytes=100 * 1024 * 1024,
        ),
    )(x)
    return out.reshape(B, S, D)
